# baseline (speedup 1.0000x reference)
import functools
import math

import jax
import jax.numpy as jnp
import numpy as np
from jax import lax
from jax.experimental import pallas as pl
from jax.experimental.pallas import tpu as pltpu
from jax.experimental.pallas import tpu_sc as plsc

_Rcr = 5.2
_Rca = 3.5
_EtaR = 16.0
_EtaA = 8.0
_ShfR = [0.9, 1.16875, 1.4375, 1.70625, 1.975, 2.24375, 2.5125, 2.78125,
         3.05, 3.31875, 3.5875, 3.85625, 4.125, 4.39375, 4.6625, 4.93125]
_ShfA = [0.9, 1.55, 2.2, 2.85]
_ShfZ = np.array([0.19634954, 0.58904862, 0.9817477, 1.3744468, 1.7671459,
                  2.1598449, 2.552544, 2.9452431], dtype=np.float64)
_COSZ = np.cos(_ShfZ).astype(np.float32).tolist()
_SINZ = np.sin(_ShfZ).astype(np.float32).tolist()
_C, _A = 32, 32
_NS = 4
_NSP = 10

_ygrid = np.linspace(0.0, math.pi ** 2, 4096)
_cheb = np.polynomial.Chebyshev.fit(_ygrid, np.cos(np.sqrt(_ygrid)), 10,
                                    domain=[0.0, math.pi ** 2])
_COSC = _cheb.convert(kind=np.polynomial.Polynomial).coef.astype(
    np.float32).tolist()


def _cos_poly(y):
    acc = jnp.full((16,), _COSC[-1], jnp.float32)
    for c in reversed(_COSC[:-1]):
        acc = acc * y + c
    return acc


def _rsqrt(x):
    i = plsc.bitcast(x, jnp.int32)
    i = 0x5F3759DF - lax.shift_right_logical(i, 1)
    y = plsc.bitcast(i, jnp.float32)
    for _ in range(3):
        y = y * (1.5 - 0.5 * x * y * y)
    return y


def _iota16():
    return lax.broadcasted_iota(jnp.int32, (16,), 0)


def _sc_body(ct_hbm, sp_hbm, outr_hbm, outa_hbm,
             ct_v, sp_v,
             dx_v, dy_v, dz_v, dd_v, di_v, fca_v, fcr_v, pt_v,
             acc2_v, accr2_v, accr_v, acca_v, sem):
    m = lax.axis_index("c") * 16 + lax.axis_index("s")

    pltpu.sync_copy(ct_hbm.at[m], ct_v)
    pltpu.sync_copy(sp_hbm.at[m], sp_v)

    zero16 = jnp.zeros((16,), jnp.float32)

    @plsc.parallel_loop(0, 640)
    def _z(i):
        acc2_v[pl.ds(i * 16, 16)] = zero16

        @pl.when(i < 128)
        def _():
            accr2_v[pl.ds(i * 16, 16)] = zero16

    @plsc.parallel_loop(0, 32)
    def _tab(a):
        av = jnp.broadcast_to(a, (16,))
        cxa = plsc.load_gather(ct_v, [av])
        cya = plsc.load_gather(ct_v, [av + 32])
        cza = plsc.load_gather(ct_v, [av + 64])
        for c in range(2):
            jv = c * 16 + _iota16()
            base = a * 32 + c * 16
            dxc = cxa - ct_v[pl.ds(c * 16, 16)]
            dyc = cya - ct_v[pl.ds(32 + c * 16, 16)]
            dzc = cza - ct_v[pl.ds(64 + c * 16, 16)]
            d2 = dxc * dxc + dyc * dyc + dzc * dzc
            rs = _rsqrt(d2)
            dist = d2 * rs
            offd = jv != av
            yr = d2 * (math.pi / _Rcr) ** 2
            fcr = 0.125 * _cos_poly(yr) + 0.125
            mr = (dist <= _Rcr) & offd
            fcr = jnp.where(mr, fcr, zero16)
            ya = d2 * (math.pi / _Rca) ** 2
            fca = 0.5 * _cos_poly(ya) + 0.5
            ma = (dist <= _Rca) & offd
            fca = jnp.where(ma, 1.4142135623730951 * fca, zero16)
            dx_v[pl.ds(base, 16)] = dxc
            dy_v[pl.ds(base, 16)] = dyc
            dz_v[pl.ds(base, 16)] = dzc
            dd_v[pl.ds(base, 16)] = dist
            di_v[pl.ds(base, 16)] = rs
            fca_v[pl.ds(base, 16)] = fca
            fcr_v[pl.ds(base, 16)] = fcr

    @plsc.parallel_loop(0, 32)
    def _ptab(j):
        spjv = plsc.load_gather(sp_v, [jnp.broadcast_to(j, (16,))])
        for c in range(2):
            spk = sp_v[pl.ds(c * 16, 16)]
            mn = jnp.minimum(spjv, spk)
            mx = jnp.maximum(spjv, spk)
            pidx = (mn * _NS - lax.shift_right_logical(mn * (mn + 1), 1)
                    + mx)
            pt_v[pl.ds(j * 32 + c * 16, 16)] = lax.shift_left(pidx, 10)

    a_vec = [c * 16 + _iota16() for c in range(2)]
    a32 = [lax.shift_left(v, 5) for v in a_vec]

    @plsc.parallel_loop(0, 32)
    def _rad(j):
        spjv = plsc.load_gather(sp_v, [jnp.broadcast_to(j, (16,))])
        sbase = lax.shift_left(spjv[0], 9)
        for c in range(2):
            ij = a32[c] + j
            d = plsc.load_gather(dd_v, [ij])
            fr = plsc.load_gather(fcr_v, [ij])
            for f in range(16):
                w = d - _ShfR[f]
                val = jnp.exp(w * w * (-_EtaR)) * fr
                plsc.addupdate(accr2_v.at[pl.ds(sbase + f * 32 + c * 16, 16)],
                               val)

    def _ango(j, _):
        @plsc.parallel_loop(0, j)
        def _angi(k):
            pv = plsc.load_gather(pt_v, [jnp.broadcast_to(j * 32 + k, (16,))])
            pbase = pv[0]
            for c in range(2):
                ij = a32[c] + j
                ik = a32[c] + k
                d1 = plsc.load_gather(dd_v, [ij])
                d2g = plsc.load_gather(dd_v, [ik])
                r1 = plsc.load_gather(di_v, [ij])
                r2 = plsc.load_gather(di_v, [ik])
                fj = plsc.load_gather(fca_v, [ij])
                fk = plsc.load_gather(fca_v, [ik])
                dxj = plsc.load_gather(dx_v, [ij])
                dxk = plsc.load_gather(dx_v, [ik])
                dyj = plsc.load_gather(dy_v, [ij])
                dyk = plsc.load_gather(dy_v, [ik])
                dzj = plsc.load_gather(dz_v, [ij])
                dzk = plsc.load_gather(dz_v, [ik])
                fcp = fj * fk
                dots = dxj * dxk + dyj * dyk + dzj * dzk
                cc = 0.95 * dots * (r1 * r2)
                x = 1.0 - cc * cc
                ss = x * _rsqrt(x)
                u = 0.5 * cc
                v = 0.5 * ss
                avg = 0.5 * (d1 + d2g)
                gs = []
                for s in range(4):
                    w = avg - _ShfA[s]
                    gs.append(jnp.exp(w * w * (-_EtaA)) * fcp)
                t32 = []
                for z in range(8):
                    t = 0.5 + _COSZ[z] * u + _SINZ[z] * v
                    t = t * t
                    t = t * t
                    t = t * t
                    t = t * t
                    t = t * t
                    t32.append(t)
                for s in range(4):
                    for z in range(8):
                        plsc.addupdate(
                            acc2_v.at[pl.ds(pbase + (s * 8 + z) * 32 + c * 16,
                                            16)],
                            gs[s] * t32[z])
        return ()

    lax.fori_loop(1, 32, _ango, ())

    f32s = _iota16() * 32

    @plsc.parallel_loop(0, 32)
    def _tr(a):
        for p in range(_NSP):
            for c in range(2):
                src = plsc.load_gather(
                    acc2_v, [p * 1024 + (c * 16) * 32 + f32s + a])
                acca_v[pl.ds(a * 320 + p * 32 + c * 16, 16)] = src
        for s in range(_NS):
            srcr = plsc.load_gather(accr2_v, [s * 512 + f32s + a])
            accr_v[pl.ds(a * 64 + s * 16, 16)] = srcr

    pltpu.sync_copy(accr_v, outr_hbm.at[m])
    pltpu.sync_copy(acca_v, outa_hbm.at[m])


@functools.partial(jax.jit, static_argnums=())
def _sc_call(ct, sp):
    mesh = plsc.VectorSubcoreMesh(core_axis_name="c", subcore_axis_name="s")
    f = pl.kernel(
        _sc_body,
        out_type=(jax.ShapeDtypeStruct((_C, _A * _NS * 16), jnp.float32),
                  jax.ShapeDtypeStruct((_C, _A * _NSP * 32), jnp.float32)),
        mesh=mesh,
        compiler_params=pltpu.CompilerParams(needs_layout_passes=False),
        scratch_types=[
            pltpu.VMEM((96,), jnp.float32),
            pltpu.VMEM((_A,), jnp.int32),
            pltpu.VMEM((_A * _A,), jnp.float32),
            pltpu.VMEM((_A * _A,), jnp.float32),
            pltpu.VMEM((_A * _A,), jnp.float32),
            pltpu.VMEM((_A * _A,), jnp.float32),
            pltpu.VMEM((_A * _A,), jnp.float32),
            pltpu.VMEM((_A * _A,), jnp.float32),
            pltpu.VMEM((_A * _A,), jnp.float32),
            pltpu.VMEM((_A * _A,), jnp.int32),
            pltpu.VMEM((_NSP * 32 * _A,), jnp.float32),
            pltpu.VMEM((_NS * 16 * _A,), jnp.float32),
            pltpu.VMEM((_A * _NS * 16,), jnp.float32),
            pltpu.VMEM((_A * _NSP * 32,), jnp.float32),
            pltpu.SemaphoreType.DMA,
        ],
    )
    return f(ct, sp)


def kernel(species, coordinates):
    ct = jnp.transpose(coordinates, (0, 2, 1)).reshape(_C, 96)
    sp = species.astype(jnp.int32)
    outr, outa = _sc_call(ct, sp)
    rad = outr.reshape(_C, _A, _NS * 16)
    ang = outa.reshape(_C, _A, _NSP * 32)
    return jnp.concatenate([rad, ang], axis=-1)


# speedup vs baseline: 1.2196x; 1.2196x over previous
import functools
import math

import jax
import jax.numpy as jnp
import numpy as np
from jax import lax
from jax.experimental import pallas as pl
from jax.experimental.pallas import tpu as pltpu
from jax.experimental.pallas import tpu_sc as plsc

_Rcr = 5.2
_Rca = 3.5
_EtaR = 16.0
_EtaA = 8.0
_ShfR = [0.9, 1.16875, 1.4375, 1.70625, 1.975, 2.24375, 2.5125, 2.78125,
         3.05, 3.31875, 3.5875, 3.85625, 4.125, 4.39375, 4.6625, 4.93125]
_ShfA = [0.9, 1.55, 2.2, 2.85]
_ShfZ = np.array([0.19634954, 0.58904862, 0.9817477, 1.3744468, 1.7671459,
                  2.1598449, 2.552544, 2.9452431], dtype=np.float64)
_COSZ = np.cos(_ShfZ).astype(np.float32).tolist()
_SINZ = np.sin(_ShfZ).astype(np.float32).tolist()
_C, _A = 32, 32
_NS = 4
_NSP = 10

_ygrid = np.linspace(0.0, math.pi ** 2, 4096)
_cheb = np.polynomial.Chebyshev.fit(_ygrid, np.cos(np.sqrt(_ygrid)), 10,
                                    domain=[0.0, math.pi ** 2])
_COSC = _cheb.convert(kind=np.polynomial.Polynomial).coef.astype(
    np.float32).tolist()


def _cos_poly(y):
    acc = jnp.full((16,), _COSC[-1], jnp.float32)
    for c in reversed(_COSC[:-1]):
        acc = acc * y + c
    return acc


def _rsqrt(x):
    i = plsc.bitcast(x, jnp.int32)
    i = 0x5F3759DF - lax.shift_right_logical(i, 1)
    y = plsc.bitcast(i, jnp.float32)
    for _ in range(3):
        y = y * (1.5 - 0.5 * x * y * y)
    return y


def _iota16():
    return lax.broadcasted_iota(jnp.int32, (16,), 0)


def _sc_body(ct_hbm, sp_hbm, outr_hbm, outa_hbm,
             ct_v, sp_v,
             dx_v, dy_v, dz_v, dd_v, di_v, fca_v, fcr_v, pt_v,
             acc2_v, accr2_v, accr_v, acca_v, sem):
    m = lax.axis_index("c") * 16 + lax.axis_index("s")

    pltpu.sync_copy(ct_hbm.at[m], ct_v)
    pltpu.sync_copy(sp_hbm.at[m], sp_v)

    zero16 = jnp.zeros((16,), jnp.float32)

    @plsc.parallel_loop(0, 640)
    def _z(i):
        acc2_v[pl.ds(i * 16, 16)] = zero16

        @pl.when(i < 128)
        def _():
            accr2_v[pl.ds(i * 16, 16)] = zero16

    @plsc.parallel_loop(0, 32)
    def _tab(a):
        av = jnp.broadcast_to(a, (16,))
        cxa = plsc.load_gather(ct_v, [av])
        cya = plsc.load_gather(ct_v, [av + 32])
        cza = plsc.load_gather(ct_v, [av + 64])
        for c in range(2):
            jv = c * 16 + _iota16()
            base = a * 32 + c * 16
            dxc = cxa - ct_v[pl.ds(c * 16, 16)]
            dyc = cya - ct_v[pl.ds(32 + c * 16, 16)]
            dzc = cza - ct_v[pl.ds(64 + c * 16, 16)]
            d2 = dxc * dxc + dyc * dyc + dzc * dzc
            rs = _rsqrt(d2)
            dist = d2 * rs
            offd = jv != av
            yr = d2 * (math.pi / _Rcr) ** 2
            fcr = 0.125 * _cos_poly(yr) + 0.125
            mr = (dist <= _Rcr) & offd
            fcr = jnp.where(mr, fcr, zero16)
            ya = d2 * (math.pi / _Rca) ** 2
            fca = 0.5 * _cos_poly(ya) + 0.5
            ma = (dist <= _Rca) & offd
            fca = jnp.where(ma, 1.4142135623730951 * fca, zero16)
            dx_v[pl.ds(base, 16)] = dxc
            dy_v[pl.ds(base, 16)] = dyc
            dz_v[pl.ds(base, 16)] = dzc
            dd_v[pl.ds(base, 16)] = dist
            di_v[pl.ds(base, 16)] = rs
            fca_v[pl.ds(base, 16)] = fca
            fcr_v[pl.ds(base, 16)] = fcr

    @plsc.parallel_loop(0, 32)
    def _ptab(j):
        spjv = plsc.load_gather(sp_v, [jnp.broadcast_to(j, (16,))])
        for c in range(2):
            spk = sp_v[pl.ds(c * 16, 16)]
            mn = jnp.minimum(spjv, spk)
            mx = jnp.maximum(spjv, spk)
            pidx = (mn * _NS - lax.shift_right_logical(mn * (mn + 1), 1)
                    + mx)
            pt_v[pl.ds(j * 32 + c * 16, 16)] = lax.shift_left(pidx, 10)

    a_vec = [c * 16 + _iota16() for c in range(2)]
    a32 = [lax.shift_left(v, 5) for v in a_vec]

    @plsc.parallel_loop(0, 32)
    def _rad(j):
        spjv = plsc.load_gather(sp_v, [jnp.broadcast_to(j, (16,))])
        sbase = lax.shift_left(spjv[0], 9)
        for c in range(2):
            ij = a32[c] + j
            d = plsc.load_gather(dd_v, [ij])
            fr = plsc.load_gather(fcr_v, [ij])
            for f in range(16):
                w = d - _ShfR[f]
                val = jnp.exp(w * w * (-_EtaR)) * fr
                plsc.addupdate(accr2_v.at[pl.ds(sbase + f * 32 + c * 16, 16)],
                               val)

    def _ango(j, _):
        jside = []
        for c in range(2):
            ij = a32[c] + j
            jside.append((plsc.load_gather(dd_v, [ij]),
                          plsc.load_gather(di_v, [ij]),
                          plsc.load_gather(fca_v, [ij]),
                          plsc.load_gather(dx_v, [ij]),
                          plsc.load_gather(dy_v, [ij]),
                          plsc.load_gather(dz_v, [ij])))

        @plsc.parallel_loop(0, j)
        def _angi(k):
            pv = plsc.load_gather(pt_v, [jnp.broadcast_to(j * 32 + k, (16,))])
            pbase = pv[0]
            for c in range(2):
                ik = a32[c] + k
                d1, r1, fj, dxj, dyj, dzj = jside[c]
                d2g = plsc.load_gather(dd_v, [ik])
                r2 = plsc.load_gather(di_v, [ik])
                fk = plsc.load_gather(fca_v, [ik])
                dxk = plsc.load_gather(dx_v, [ik])
                dyk = plsc.load_gather(dy_v, [ik])
                dzk = plsc.load_gather(dz_v, [ik])
                fcp = fj * fk
                dots = dxj * dxk + dyj * dyk + dzj * dzk
                cc = 0.95 * dots * (r1 * r2)
                x = 1.0 - cc * cc
                ss = x * _rsqrt(x)
                u = 0.5 * cc
                v = 0.5 * ss
                avg = 0.5 * (d1 + d2g)
                gs = []
                for s in range(4):
                    w = avg - _ShfA[s]
                    gs.append(jnp.exp(w * w * (-_EtaA)) * fcp)
                t32 = []
                for z in range(8):
                    t = 0.5 + _COSZ[z] * u + _SINZ[z] * v
                    t = t * t
                    t = t * t
                    t = t * t
                    t = t * t
                    t = t * t
                    t32.append(t)
                for s in range(4):
                    for z in range(8):
                        plsc.addupdate(
                            acc2_v.at[pl.ds(pbase + (s * 8 + z) * 32 + c * 16,
                                            16)],
                            gs[s] * t32[z])
        return ()

    lax.fori_loop(1, 32, _ango, ())

    f32s = _iota16() * 32

    @plsc.parallel_loop(0, 32)
    def _tr(a):
        for p in range(_NSP):
            for c in range(2):
                src = plsc.load_gather(
                    acc2_v, [p * 1024 + (c * 16) * 32 + f32s + a])
                acca_v[pl.ds(a * 320 + p * 32 + c * 16, 16)] = src
        for s in range(_NS):
            srcr = plsc.load_gather(accr2_v, [s * 512 + f32s + a])
            accr_v[pl.ds(a * 64 + s * 16, 16)] = srcr

    pltpu.sync_copy(accr_v, outr_hbm.at[m])
    pltpu.sync_copy(acca_v, outa_hbm.at[m])


@functools.partial(jax.jit, static_argnums=())
def _sc_call(ct, sp):
    mesh = plsc.VectorSubcoreMesh(core_axis_name="c", subcore_axis_name="s")
    f = pl.kernel(
        _sc_body,
        out_type=(jax.ShapeDtypeStruct((_C, _A * _NS * 16), jnp.float32),
                  jax.ShapeDtypeStruct((_C, _A * _NSP * 32), jnp.float32)),
        mesh=mesh,
        compiler_params=pltpu.CompilerParams(needs_layout_passes=False),
        scratch_types=[
            pltpu.VMEM((96,), jnp.float32),
            pltpu.VMEM((_A,), jnp.int32),
            pltpu.VMEM((_A * _A,), jnp.float32),
            pltpu.VMEM((_A * _A,), jnp.float32),
            pltpu.VMEM((_A * _A,), jnp.float32),
            pltpu.VMEM((_A * _A,), jnp.float32),
            pltpu.VMEM((_A * _A,), jnp.float32),
            pltpu.VMEM((_A * _A,), jnp.float32),
            pltpu.VMEM((_A * _A,), jnp.float32),
            pltpu.VMEM((_A * _A,), jnp.int32),
            pltpu.VMEM((_NSP * 32 * _A,), jnp.float32),
            pltpu.VMEM((_NS * 16 * _A,), jnp.float32),
            pltpu.VMEM((_A * _NS * 16,), jnp.float32),
            pltpu.VMEM((_A * _NSP * 32,), jnp.float32),
            pltpu.SemaphoreType.DMA,
        ],
    )
    return f(ct, sp)


def kernel(species, coordinates):
    ct = jnp.transpose(coordinates, (0, 2, 1)).reshape(_C, 96)
    sp = species.astype(jnp.int32)
    outr, outa = _sc_call(ct, sp)
    rad = outr.reshape(_C, _A, _NS * 16)
    ang = outa.reshape(_C, _A, _NSP * 32)
    return jnp.concatenate([rad, ang], axis=-1)
